# baseline (speedup 1.0000x reference)
import functools
import math

import jax
import jax.numpy as jnp
from jax.experimental import pallas as pl
from jax.experimental.pallas import tpu as pltpu

_SIGMA = 0.5
_K = 16
_NKP = 15


def _layer_body(diff_ref, nf_ref, kp_ref, kw_ref, b_ref, out_ref, *, cin, cout, nt):
    diff = diff_ref[0]
    nf = nf_ref[0]
    kp = kp_ref[...]
    sq = jnp.zeros((nt, _K, _NKP), jnp.float32)
    for d in range(3):
        delta = diff[:, :, d][:, :, None] - kp[:, d][None, None, :]
        sq = sq + delta * delta
    infl = jnp.exp(sq * (-1.0 / (_SIGMA * _SIGMA)))
    acc = jnp.zeros((nt, _NKP, cin), jnp.float32)
    for k in range(_K):
        acc = acc + infl[:, k, :][:, :, None] * nf[:, k, :][:, None, :]
    out = jnp.zeros((nt, cout), jnp.float32)
    for i in range(_NKP):
        out = out + jnp.dot(acc[:, i, :], kw_ref[i],
                            preferred_element_type=jnp.float32)
    out = out + b_ref[...][None, :]
    out_ref[0] = jnp.maximum(out, 0.0)


def _kp_layer(diff, nf, kp, kw, b):
    B, N, _, cin = nf.shape
    cout = kw.shape[-1]
    nt = 256
    grid = (B, N // nt)
    body = functools.partial(_layer_body, cin=cin, cout=cout, nt=nt)
    return pl.pallas_call(
        body,
        grid=grid,
        in_specs=[
            pl.BlockSpec((1, nt, _K, 3), lambda b_, n: (b_, n, 0, 0)),
            pl.BlockSpec((1, nt, _K, cin), lambda b_, n: (b_, n, 0, 0)),
            pl.BlockSpec((_NKP, 3), lambda b_, n: (0, 0)),
            pl.BlockSpec((_NKP, cin, cout), lambda b_, n: (0, 0, 0)),
            pl.BlockSpec((cout,), lambda b_, n: (0,)),
        ],
        out_specs=pl.BlockSpec((1, nt, cout), lambda b_, n: (b_, n, 0)),
        out_shape=jax.ShapeDtypeStruct((B, N, cout), jnp.float32),
    )(diff, nf, kp, kw, b)


def kernel(coords, features, kp1, kw1, b1, kp2, kw2, b2, kp3, kw3, b3, kp4, kw4, b4):
    X = jnp.transpose(coords, (0, 2, 1))
    feat = jnp.transpose(features, (0, 2, 1))

    inner = -2.0 * jnp.einsum('bnd,bmd->bnm', X, X)
    xx = jnp.sum(X * X, axis=-1)
    pd = -xx[:, :, None] - inner - xx[:, None, :]
    nbr_idx = jax.lax.top_k(pd, _K)[1]

    gather = jax.vmap(lambda a, i: a[i])
    nc = gather(X, nbr_idx)
    diff = nc - X[:, :, None, :]

    outs = []
    f = feat
    for (kp, kw, b) in ((kp1, kw1, b1), (kp2, kw2, b2), (kp3, kw3, b3), (kp4, kw4, b4)):
        nf = gather(f, nbr_idx)
        f = _kp_layer(diff, nf, kp, kw, b)
        outs.append(f)
    xc = jnp.concatenate(outs, axis=-1)
    return jnp.transpose(xc, (0, 2, 1))


# speedup vs baseline: 1.5012x; 1.5012x over previous
import functools
import math

import jax
import jax.numpy as jnp
from jax.experimental import pallas as pl
from jax.experimental.pallas import tpu as pltpu

_SIGMA = 0.5
_K = 16
_NKP = 15


def _layer_body(diff_ref, nf_ref, kp_ref, kw_ref, b_ref, out_ref, *, cin, cout, nt):
    diff = diff_ref[0]
    nf = nf_ref[0]
    kp = kp_ref[...]
    sq = jnp.zeros((nt, _K, _NKP), jnp.float32)
    for d in range(3):
        delta = diff[:, :, d][:, :, None] - kp[:, d][None, None, :]
        sq = sq + delta * delta
    infl = jnp.exp(sq * (-1.0 / (_SIGMA * _SIGMA)))
    acc = jnp.zeros((nt, _NKP, cin), jnp.float32)
    for k in range(_K):
        acc = acc + infl[:, k, :][:, :, None] * nf[:, k, :][:, None, :]
    out = jnp.zeros((nt, cout), jnp.float32)
    for i in range(_NKP):
        out = out + jnp.dot(acc[:, i, :], kw_ref[i],
                            preferred_element_type=jnp.float32)
    out = out + b_ref[...][None, :]
    out_ref[0] = jnp.maximum(out, 0.0)


def _topk_body(cb_ref, ck_ref, out_ref, *, r, n):
    cb = cb_ref[0]
    ck = ck_ref[0]
    dots = jax.lax.dot_general(cb, ck, (((0,), (0,)), ((), ())),
                               preferred_element_type=jnp.float32)
    xx = jnp.sum(ck * ck, axis=0)[None, :]
    vals = 2.0 * dots - xx
    iota = jax.lax.broadcasted_iota(jnp.int32, (r, n), 1)
    cols = []
    for _ in range(_K):
        m = jnp.max(vals, axis=1, keepdims=True)
        eq = vals == m
        idx = jnp.min(jnp.where(eq, iota, n), axis=1, keepdims=True)
        cols.append(idx)
        vals = jnp.where(iota == idx, -jnp.inf, vals)
    out_ref[0] = jnp.concatenate(cols, axis=1)


def _topk_idx(coords):
    B, _, N = coords.shape
    r = 256
    body = functools.partial(_topk_body, r=r, n=N)
    return pl.pallas_call(
        body,
        grid=(B, N // r),
        in_specs=[
            pl.BlockSpec((1, 3, r), lambda b_, t: (b_, 0, t)),
            pl.BlockSpec((1, 3, N), lambda b_, t: (b_, 0, 0)),
        ],
        out_specs=pl.BlockSpec((1, r, _K), lambda b_, t: (b_, t, 0)),
        out_shape=jax.ShapeDtypeStruct((B, N, _K), jnp.int32),
    )(coords, coords)


def _kp_layer(diff, nf, kp, kw, b):
    B, N, _, cin = nf.shape
    cout = kw.shape[-1]
    nt = 256
    grid = (B, N // nt)
    body = functools.partial(_layer_body, cin=cin, cout=cout, nt=nt)
    return pl.pallas_call(
        body,
        grid=grid,
        in_specs=[
            pl.BlockSpec((1, nt, _K, 3), lambda b_, n: (b_, n, 0, 0)),
            pl.BlockSpec((1, nt, _K, cin), lambda b_, n: (b_, n, 0, 0)),
            pl.BlockSpec((_NKP, 3), lambda b_, n: (0, 0)),
            pl.BlockSpec((_NKP, cin, cout), lambda b_, n: (0, 0, 0)),
            pl.BlockSpec((cout,), lambda b_, n: (0,)),
        ],
        out_specs=pl.BlockSpec((1, nt, cout), lambda b_, n: (b_, n, 0)),
        out_shape=jax.ShapeDtypeStruct((B, N, cout), jnp.float32),
    )(diff, nf, kp, kw, b)


def kernel(coords, features, kp1, kw1, b1, kp2, kw2, b2, kp3, kw3, b3, kp4, kw4, b4):
    X = jnp.transpose(coords, (0, 2, 1))
    feat = jnp.transpose(features, (0, 2, 1))

    nbr_idx = _topk_idx(coords)

    gather = jax.vmap(lambda a, i: a[i])
    nc = gather(X, nbr_idx)
    diff = nc - X[:, :, None, :]

    outs = []
    f = feat
    for (kp, kw, b) in ((kp1, kw1, b1), (kp2, kw2, b2), (kp3, kw3, b3), (kp4, kw4, b4)):
        nf = gather(f, nbr_idx)
        f = _kp_layer(diff, nf, kp, kw, b)
        outs.append(f)
    xc = jnp.concatenate(outs, axis=-1)
    return jnp.transpose(xc, (0, 2, 1))


# speedup vs baseline: 7.9583x; 5.3013x over previous
import functools
import math

import jax
import jax.numpy as jnp
from jax import lax
from jax.experimental import pallas as pl
from jax.experimental.pallas import tpu as pltpu
from jax.experimental.pallas import tpu_sc as plsc

_SIGMA = 0.5
_K = 16
_NKP = 15

_NC, _NS = 2, 16
_NW = _NC * _NS
_CH = 128


def _sc_gather(table, idx3):
    V, D = table.shape
    cpw = idx3.shape[1]
    m_per_w = cpw * _CH
    M = _NW * m_per_w
    mesh = plsc.VectorSubcoreMesh(core_axis_name="c", subcore_axis_name="s")

    @functools.partial(
        pl.kernel, mesh=mesh,
        compiler_params=pltpu.CompilerParams(use_tc_tiling_on_sc=False),
        out_type=jax.ShapeDtypeStruct((M, D), jnp.float32),
        scratch_types=[
            pltpu.VMEM((cpw, _CH), jnp.int32),
            pltpu.VMEM((_CH, D), jnp.float32),
            pltpu.SemaphoreType.DMA,
        ],
    )
    def k(table_hbm, idx_hbm, out_hbm, idx_v, rows_v, sem):
        wid = lax.axis_index("s") * _NC + lax.axis_index("c")
        base = wid * m_per_w
        pltpu.sync_copy(idx_hbm.at[wid], idx_v)

        def body(c, _):
            pltpu.async_copy(table_hbm.at[idx_v.at[c]], rows_v, sem).wait()
            pltpu.sync_copy(rows_v, out_hbm.at[pl.ds(base + c * _CH, _CH)])
            return _

        lax.fori_loop(0, cpw, body, 0)

    return k(table, idx3)


def _layer_body(nc_ref, xq_ref, nf_ref, kp_ref, kw_ref, b_ref, out_ref, *, cin, cout, nt):
    diff = nc_ref[0][:, :, :3] - xq_ref[0][:, None, :3]
    nf = nf_ref[0]
    kp = kp_ref[...]
    sq = jnp.zeros((nt, _K, _NKP), jnp.float32)
    for d in range(3):
        delta = diff[:, :, d][:, :, None] - kp[:, d][None, None, :]
        sq = sq + delta * delta
    infl = jnp.exp(sq * (-1.0 / (_SIGMA * _SIGMA)))
    acc = jnp.zeros((nt, _NKP, cin), jnp.float32)
    for k in range(_K):
        acc = acc + infl[:, k, :][:, :, None] * nf[:, k, :][:, None, :]
    out = jnp.zeros((nt, cout), jnp.float32)
    for i in range(_NKP):
        out = out + jnp.dot(acc[:, i, :], kw_ref[i],
                            preferred_element_type=jnp.float32)
    out = out + b_ref[...][None, :]
    out_ref[0] = jnp.maximum(out, 0.0)


def _topk_body(cb_ref, ck_ref, out_ref, *, r, n):
    cb = cb_ref[0]
    ck = ck_ref[0]
    dots = jax.lax.dot_general(cb, ck, (((0,), (0,)), ((), ())),
                               preferred_element_type=jnp.float32)
    xx = jnp.sum(ck * ck, axis=0)[None, :]
    vals = 2.0 * dots - xx
    iota = jax.lax.broadcasted_iota(jnp.int32, (r, n), 1)
    cols = []
    for _ in range(_K):
        m = jnp.max(vals, axis=1, keepdims=True)
        eq = vals == m
        idx = jnp.min(jnp.where(eq, iota, n), axis=1, keepdims=True)
        cols.append(idx)
        vals = jnp.where(iota == idx, -jnp.inf, vals)
    out_ref[0] = jnp.concatenate(cols, axis=1)


def _topk_idx(coords):
    B, _, N = coords.shape
    r = 256
    body = functools.partial(_topk_body, r=r, n=N)
    return pl.pallas_call(
        body,
        grid=(B, N // r),
        in_specs=[
            pl.BlockSpec((1, 3, r), lambda b_, t: (b_, 0, t)),
            pl.BlockSpec((1, 3, N), lambda b_, t: (b_, 0, 0)),
        ],
        out_specs=pl.BlockSpec((1, r, _K), lambda b_, t: (b_, t, 0)),
        out_shape=jax.ShapeDtypeStruct((B, N, _K), jnp.int32),
    )(coords, coords)


def _kp_layer(nc8, xq8, nf, kp, kw, b):
    B, N, _, cin = nf.shape
    cout = kw.shape[-1]
    nt = 256
    grid = (B, N // nt)
    body = functools.partial(_layer_body, cin=cin, cout=cout, nt=nt)
    return pl.pallas_call(
        body,
        grid=grid,
        in_specs=[
            pl.BlockSpec((1, nt, _K, 8), lambda b_, n: (b_, n, 0, 0)),
            pl.BlockSpec((1, nt, 8), lambda b_, n: (b_, n, 0)),
            pl.BlockSpec((1, nt, _K, cin), lambda b_, n: (b_, n, 0, 0)),
            pl.BlockSpec((_NKP, 3), lambda b_, n: (0, 0)),
            pl.BlockSpec((_NKP, cin, cout), lambda b_, n: (0, 0, 0)),
            pl.BlockSpec((cout,), lambda b_, n: (0,)),
        ],
        out_specs=pl.BlockSpec((1, nt, cout), lambda b_, n: (b_, n, 0)),
        out_shape=jax.ShapeDtypeStruct((B, N, cout), jnp.float32),
    )(nc8, xq8, nf, kp, kw, b)


def kernel(coords, features, kp1, kw1, b1, kp2, kw2, b2, kp3, kw3, b3, kp4, kw4, b4):
    X = jnp.transpose(coords, (0, 2, 1))
    feat = jnp.transpose(features, (0, 2, 1))

    B, N, _ = X.shape
    nbr_idx = _topk_idx(coords)

    gidx = nbr_idx + (jnp.arange(B, dtype=jnp.int32) * N)[:, None, None]
    cpw = (B * N * _K) // (_NW * _CH)
    gidx3 = gidx.reshape(_NW, cpw, _CH)

    xpad = jnp.pad(X, ((0, 0), (0, 0), (0, 5))).reshape(B * N, 8)
    xq8 = xpad.reshape(B, N, 8)
    nc8 = _sc_gather(xpad, gidx3).reshape(B, N, _K, 8)

    outs = []
    f = feat
    for (kp, kw, b) in ((kp1, kw1, b1), (kp2, kw2, b2), (kp3, kw3, b3), (kp4, kw4, b4)):
        cin = f.shape[-1]
        nf = _sc_gather(f.reshape(B * N, cin), gidx3).reshape(B, N, _K, cin)
        f = _kp_layer(nc8, xq8, nf, kp, kw, b)
        outs.append(f)
    xc = jnp.concatenate(outs, axis=-1)
    return jnp.transpose(xc, (0, 2, 1))
